# baseline (speedup 1.0000x reference)
import jax
import jax.numpy as jnp
from jax.experimental import pallas as pl
from jax.experimental.pallas import tpu as pltpu

CHARGE_DIM = 64


def kernel(sequence, charge, length):
    B, L = sequence.shape
    D = CHARGE_DIM
    LB = 2048

    def body(charge_ref, length_ref, out_ref):
        b = pl.program_id(0)
        j = pl.program_id(1)
        ch = charge_ref[b]
        ln = length_ref[b]
        pos = jax.lax.broadcasted_iota(jnp.int32, (LB, D), 0) + j * LB
        out_ref[0] = jnp.where(pos < ln, ch, jnp.float32(0.0))

    return pl.pallas_call(
        body,
        grid=(B, L // LB),
        in_specs=[
            pl.BlockSpec(memory_space=pltpu.SMEM),
            pl.BlockSpec(memory_space=pltpu.SMEM),
        ],
        out_specs=pl.BlockSpec((1, LB, D), lambda b, j: (b, j, 0)),
        out_shape=jax.ShapeDtypeStruct((B, L, D), jnp.float32),
    )(charge, length)
